# Initial kernel scaffold; baseline (speedup 1.0000x reference)
import functools

import jax
import jax.numpy as jnp
from jax import lax
from jax.experimental import pallas as pl
from jax.experimental.pallas import tpu as pltpu
from jax.experimental.pallas import tpu_sc as plsc

ND = 10000
NP = 50000
DIM = 128
E = 300000
NLAYER = 3

NTILES = 32
NBATCH = 76
EPT = NBATCH * 128
EPAD = NTILES * EPT
ROW_BLK = 400

NP_PAD = 51200
ND_PAD = 10240


def _enc_body(x_ref, w_ref, b_ref, o_ref):
    h = jnp.dot(x_ref[...], w_ref[...], preferred_element_type=jnp.float32)
    h = jnp.maximum(h + b_ref[0, :][None, :], 0.0)
    for c in range(4):
        o_ref[c] = h[:, c * 32:(c + 1) * 32]


@functools.lru_cache(None)
def _enc_call(n):
    return pl.pallas_call(
        _enc_body,
        grid=(n // ROW_BLK,),
        in_specs=[
            pl.BlockSpec((ROW_BLK, DIM), lambda i: (i, 0)),
            pl.BlockSpec((DIM, DIM), lambda i: (0, 0)),
            pl.BlockSpec((1, DIM), lambda i: (0, 0)),
        ],
        out_specs=pl.BlockSpec((4, ROW_BLK, 32), lambda i: (0, i, 0)),
        out_shape=jax.ShapeDtypeStruct((4, n, 32), jnp.float32),
    )


def _proj_body(ht_ref, ws_ref, as_ref, wd_ref, ad_ref, hs_ref, als_ref, ald_ref):
    h = jnp.concatenate([ht_ref[c] for c in range(4)], axis=1)
    hs = jnp.dot(h, ws_ref[...], preferred_element_type=jnp.float32)
    for c in range(4):
        hs_ref[c] = hs[:, c * 32:(c + 1) * 32]
    als_ref[0, :] = jnp.sum(hs * as_ref[0, :][None, :], axis=1)
    v = jnp.sum(wd_ref[...] * ad_ref[0, :][None, :], axis=1)
    ald_ref[0, :] = jnp.sum(h * v[None, :], axis=1)


@functools.lru_cache(None)
def _proj_call(n):
    nb = n // ROW_BLK
    return pl.pallas_call(
        _proj_body,
        grid=(nb,),
        in_specs=[
            pl.BlockSpec((4, ROW_BLK, 32), lambda i: (0, i, 0)),
            pl.BlockSpec((DIM, DIM), lambda i: (0, 0)),
            pl.BlockSpec((1, DIM), lambda i: (0, 0)),
            pl.BlockSpec((DIM, DIM), lambda i: (0, 0)),
            pl.BlockSpec((1, DIM), lambda i: (0, 0)),
        ],
        out_specs=[
            pl.BlockSpec((4, ROW_BLK, 32), lambda i: (0, i, 0)),
            pl.BlockSpec((1, ROW_BLK), lambda i: (i, 0)),
            pl.BlockSpec((1, ROW_BLK), lambda i: (i, 0)),
        ],
        out_shape=[
            jax.ShapeDtypeStruct((4, n, 32), jnp.float32),
            jax.ShapeDtypeStruct((nb, ROW_BLK), jnp.float32),
            jax.ShapeDtypeStruct((nb, ROW_BLK), jnp.float32),
        ],
    )


def _norm_body(acc_ref, den_ref, b_ref, o_ref, *, final):
    d = den_ref[0, :] + den_ref[1, :]
    s = 1.0 / (d + 1e-16)
    parts = []
    for c in range(4):
        oc = acc_ref[c] * s[:, None] + b_ref[0, c * 32:(c + 1) * 32][None, :]
        oc = jnp.maximum(oc, 0.0)
        if final:
            parts.append(oc)
        else:
            o_ref[c] = oc
    if final:
        o_ref[...] = jnp.concatenate(parts, axis=1)


@functools.lru_cache(None)
def _norm_call(n, n_pad, final):
    if final:
        out_spec = pl.BlockSpec((ROW_BLK, DIM), lambda i: (i, 0))
        out_shape = jax.ShapeDtypeStruct((n, DIM), jnp.float32)
    else:
        out_spec = pl.BlockSpec((4, ROW_BLK, 32), lambda i: (0, i, 0))
        out_shape = jax.ShapeDtypeStruct((4, n, 32), jnp.float32)
    return pl.pallas_call(
        functools.partial(_norm_body, final=final),
        grid=(n // ROW_BLK,),
        in_specs=[
            pl.BlockSpec((4, ROW_BLK, 32), lambda i: (0, i, 0)),
            pl.BlockSpec((2, ROW_BLK), lambda i: (0, i)),
            pl.BlockSpec((1, DIM), lambda i: (0, 0)),
        ],
        out_specs=out_spec,
        out_shape=out_shape,
    )


@functools.lru_cache(None)
def _edge_kernel(n_src, n_dst, n_dst_pad):
    mesh = plsc.VectorSubcoreMesh(core_axis_name="c", subcore_axis_name="s")
    dchunk = n_dst_pad // 16

    def body(als_hbm, ald_hbm, src2_hbm, dst2_hbm, w2_hbm, den_hbm,
             als_v, ald_v, src2_v, dst2_v, w2_v, den_sh, zero_v):
        cid = lax.axis_index("c")
        sid = lax.axis_index("s")
        wid = cid * 16 + sid
        pltpu.sync_copy(als_hbm, als_v)
        pltpu.sync_copy(ald_hbm, ald_v)
        pltpu.sync_copy(src2_hbm.at[pl.ds(wid * NBATCH, NBATCH)], src2_v)
        pltpu.sync_copy(dst2_hbm.at[pl.ds(wid * NBATCH, NBATCH)], dst2_v)

        zz = jnp.zeros((16,), jnp.float32)

        def zfill(i, carry):
            zero_v[pl.ds(i * 16, 16)] = zz
            return carry

        lax.fori_loop(0, dchunk // 16, zfill, 0)
        pltpu.sync_copy(zero_v, den_sh.at[pl.ds(sid * dchunk, dchunk)])
        plsc.subcore_barrier()

        base = wid * EPT
        lane = lax.iota(jnp.int32, 16)

        def ebody(b, carry):
            for k in range(8):
                sv = src2_v[b, pl.ds(k * 16, 16)]
                dv = dst2_v[b, pl.ds(k * 16, 16)]
                e = plsc.load_gather(als_v, [sv]) + plsc.load_gather(ald_v, [dv])
                e = jnp.where(e >= 0.0, e, 0.2 * e)
                w = jnp.exp(e)
                eid = base + b * 128 + k * 16 + lane
                w = jnp.where(eid < E, w, 0.0)
                w2_v[b, pl.ds(k * 16, 16)] = w
            pltpu.sync_copy(w2_v.at[b], den_sh.at[dst2_v.at[b]], add=True)
            return carry

        lax.fori_loop(0, NBATCH, ebody, 0)
        pltpu.sync_copy(w2_v, w2_hbm.at[pl.ds(wid * NBATCH, NBATCH)])
        plsc.subcore_barrier()
        pltpu.sync_copy(den_sh.at[pl.ds(sid * dchunk, dchunk)],
                        den_hbm.at[cid].at[pl.ds(sid * dchunk, dchunk)])

    return pl.kernel(
        body,
        out_type=[
            jax.ShapeDtypeStruct((EPAD // 128, 128), jnp.float32),
            jax.ShapeDtypeStruct((2, n_dst_pad), jnp.float32),
        ],
        mesh=mesh,
        scratch_types=[
            pltpu.VMEM((n_src,), jnp.float32),
            pltpu.VMEM((n_dst,), jnp.float32),
            pltpu.VMEM((NBATCH, 128), jnp.int32),
            pltpu.VMEM((NBATCH, 128), jnp.int32),
            pltpu.VMEM((NBATCH, 128), jnp.float32),
            pltpu.VMEM_SHARED((n_dst_pad,), jnp.float32),
            pltpu.VMEM((dchunk,), jnp.float32),
        ],
    )


@functools.lru_cache(None)
def _scatter_kernel(n_src, n_dst_pad):
    mesh = plsc.VectorSubcoreMesh(core_axis_name="c", subcore_axis_name="s")
    rows_pt = n_dst_pad // 16
    nz = rows_pt // 128

    def body(hst_hbm, src2_hbm, dst2_hbm, w2_hbm, acc_hbm,
             src2_v, dst2_v, w2_v, g0, g1, acc_sh, zrow, gsem0, gsem1):
        cid = lax.axis_index("c")
        sid = lax.axis_index("s")
        wid = cid * 16 + sid
        pltpu.sync_copy(src2_hbm.at[pl.ds(wid * NBATCH, NBATCH)], src2_v)
        pltpu.sync_copy(dst2_hbm.at[pl.ds(wid * NBATCH, NBATCH)], dst2_v)
        pltpu.sync_copy(w2_hbm.at[pl.ds(wid * NBATCH, NBATCH)], w2_v)

        zz = jnp.zeros((16,), jnp.float32)

        def zfill(i, carry):
            zrow[i // 2, pl.ds((i % 2) * 16, 16)] = zz
            return carry

        lax.fori_loop(0, 256, zfill, 0)

        gbufs = (g0, g1)
        gsems = (gsem0, gsem1)

        for cc in range(2):
            c = cid * 2 + cc
            tab = hst_hbm.at[c]

            def zloop(i, carry):
                pltpu.sync_copy(zrow, acc_sh.at[pl.ds(sid * rows_pt + i * 128, 128)])
                return carry

            lax.fori_loop(0, nz, zloop, 0)
            plsc.subcore_barrier()

            pltpu.async_copy(tab.at[src2_v.at[0]], g0, gsem0)
            pltpu.async_copy(tab.at[src2_v.at[1]], g1, gsem1)

            def step(j, b):
                g = gbufs[j]
                sem = gsems[j]
                pltpu.make_async_copy(tab.at[src2_v.at[b]], g, sem).wait()
                for k in range(8):
                    wv = w2_v[b, pl.ds(k * 16, 16)]
                    for t in range(16):
                        ew = jnp.full((16,), wv[t])
                        row = k * 16 + t
                        g[row, pl.ds(0, 16)] = g[row, pl.ds(0, 16)] * ew
                        g[row, pl.ds(16, 16)] = g[row, pl.ds(16, 16)] * ew
                pltpu.sync_copy(g, acc_sh.at[dst2_v.at[b]], add=True)

                @pl.when(b + 2 < NBATCH)
                def _():
                    pltpu.async_copy(tab.at[src2_v.at[b + 2]], g, sem)

            def sbody(i, carry):
                step(0, 2 * i)
                step(1, 2 * i + 1)
                return carry

            lax.fori_loop(0, NBATCH // 2, sbody, 0)
            plsc.subcore_barrier()
            pltpu.sync_copy(acc_sh.at[pl.ds(sid * rows_pt, rows_pt)],
                            acc_hbm.at[c].at[pl.ds(sid * rows_pt, rows_pt)])

    return pl.kernel(
        body,
        out_type=jax.ShapeDtypeStruct((4, n_dst_pad, 32), jnp.float32),
        mesh=mesh,
        scratch_types=[
            pltpu.VMEM((NBATCH, 128), jnp.int32),
            pltpu.VMEM((NBATCH, 128), jnp.int32),
            pltpu.VMEM((NBATCH, 128), jnp.float32),
            pltpu.VMEM((128, 32), jnp.float32),
            pltpu.VMEM((128, 32), jnp.float32),
            pltpu.VMEM_SHARED((n_dst_pad, 32), jnp.float32),
            pltpu.VMEM((128, 32), jnp.float32),
            pltpu.SemaphoreType.DMA,
            pltpu.SemaphoreType.DMA,
        ],
    )


def kernel(x_drug, x_protein, Wenc_d, benc_d, Wenc_p, benc_p, Wsrc, Wdst,
           Asrc, Adst, Bconv, edge_index_drug_binds_protein,
           edge_index_protein_rev_binds_drug):
    def prep_edges(ei, n_src, n_dst):
        src = ei[0].astype(jnp.int32)
        dst = ei[1].astype(jnp.int32)
        pad = jnp.arange(EPAD - E, dtype=jnp.int32)
        src = jnp.concatenate([src, pad % n_src]).reshape(EPAD // 128, 128)
        dst = jnp.concatenate([dst, pad % n_dst]).reshape(EPAD // 128, 128)
        return src, dst

    src_dp, dst_dp = prep_edges(edge_index_drug_binds_protein, ND, NP)
    src_pd, dst_pd = prep_edges(edge_index_protein_rev_binds_drug, NP, ND)

    hdT = _enc_call(ND)(x_drug, Wenc_d, benc_d.reshape(1, DIM))
    hpT = _enc_call(NP)(x_protein, Wenc_p, benc_p.reshape(1, DIM))

    for l in range(NLAYER):
        hsT_dp, als_dp2, ald_pd2 = _proj_call(ND)(
            hdT, Wsrc[l, 0], Asrc[l, 0].reshape(1, DIM),
            Wdst[l, 1], Adst[l, 1].reshape(1, DIM))
        hsT_pd, als_pd2, ald_dp2 = _proj_call(NP)(
            hpT, Wsrc[l, 1], Asrc[l, 1].reshape(1, DIM),
            Wdst[l, 0], Adst[l, 0].reshape(1, DIM))
        als_dp = als_dp2.reshape(ND)
        ald_dp = ald_dp2.reshape(NP)
        als_pd = als_pd2.reshape(NP)
        ald_pd = ald_pd2.reshape(ND)

        w_dp, den_dp = _edge_kernel(ND, NP, NP_PAD)(als_dp, ald_dp, src_dp, dst_dp)
        w_pd, den_pd = _edge_kernel(NP, ND, ND_PAD)(als_pd, ald_pd, src_pd, dst_pd)

        acc_dp = _scatter_kernel(ND, NP_PAD)(hsT_dp, src_dp, dst_dp, w_dp)
        acc_pd = _scatter_kernel(NP, ND_PAD)(hsT_pd, src_pd, dst_pd, w_pd)

        final = l == NLAYER - 1
        hpT = _norm_call(NP, NP_PAD, final)(acc_dp, den_dp, Bconv[l, 0].reshape(1, DIM))
        hdT = _norm_call(ND, ND_PAD, final)(acc_pd, den_pd, Bconv[l, 1].reshape(1, DIM))

    return hdT, hpT


# speedup vs baseline: 24.1727x; 24.1727x over previous
import functools

import jax
import jax.numpy as jnp
from jax import lax
from jax.experimental import pallas as pl
from jax.experimental.pallas import tpu as pltpu
from jax.experimental.pallas import tpu_sc as plsc

ND = 10000
NP = 50000
DIM = 128
E = 300000
NLAYER = 3

NBATCH = 80
EPT = NBATCH * 128
EPAD = 32 * EPT
NBATCH16 = 2 * NBATCH
ROW_BLK = 512

NP_PAD = 51200
ND_PAD = 10240


def _enc_body(x_ref, w_ref, b_ref, o_ref):
    h = jnp.dot(x_ref[...], w_ref[...], preferred_element_type=jnp.float32)
    o_ref[...] = jnp.maximum(h + b_ref[0, :][None, :], 0.0)


@functools.lru_cache(None)
def _enc_call(n):
    return pl.pallas_call(
        _enc_body,
        grid=(pl.cdiv(n, ROW_BLK),),
        in_specs=[
            pl.BlockSpec((ROW_BLK, DIM), lambda i: (i, 0)),
            pl.BlockSpec((DIM, DIM), lambda i: (0, 0)),
            pl.BlockSpec((1, DIM), lambda i: (0, 0)),
        ],
        out_specs=pl.BlockSpec((ROW_BLK, DIM), lambda i: (i, 0)),
        out_shape=jax.ShapeDtypeStruct((n, DIM), jnp.float32),
    )


def _proj_body(h_ref, ws_ref, as_ref, wd_ref, ad_ref, hs_ref, als_ref,
               ald_ref, *, split16):
    h = h_ref[...]
    hs = jnp.dot(h, ws_ref[...], preferred_element_type=jnp.float32)
    if split16:
        for c in range(8):
            hs_ref[c] = hs[:, c * 16:(c + 1) * 16]
    else:
        hs_ref[...] = hs
    als_ref[:, 0] = jnp.sum(hs * as_ref[0, :][None, :], axis=1)
    v = jnp.sum(wd_ref[...] * ad_ref[0, :][None, :], axis=1)
    ald_ref[:, 0] = jnp.sum(h * v[None, :], axis=1)


@functools.lru_cache(None)
def _proj_call(n, split16):
    if split16:
        hs_spec = pl.BlockSpec((8, ROW_BLK, 16), lambda i: (0, i, 0))
        hs_shape = jax.ShapeDtypeStruct((8, n, 16), jnp.float32)
    else:
        hs_spec = pl.BlockSpec((ROW_BLK, DIM), lambda i: (i, 0))
        hs_shape = jax.ShapeDtypeStruct((n, DIM), jnp.float32)
    return pl.pallas_call(
        functools.partial(_proj_body, split16=split16),
        grid=(pl.cdiv(n, ROW_BLK),),
        in_specs=[
            pl.BlockSpec((ROW_BLK, DIM), lambda i: (i, 0)),
            pl.BlockSpec((DIM, DIM), lambda i: (0, 0)),
            pl.BlockSpec((1, DIM), lambda i: (0, 0)),
            pl.BlockSpec((DIM, DIM), lambda i: (0, 0)),
            pl.BlockSpec((1, DIM), lambda i: (0, 0)),
        ],
        out_specs=[
            hs_spec,
            pl.BlockSpec((ROW_BLK, 1), lambda i: (i, 0)),
            pl.BlockSpec((ROW_BLK, 1), lambda i: (i, 0)),
        ],
        out_shape=[
            hs_shape,
            jax.ShapeDtypeStruct((n, 1), jnp.float32),
            jax.ShapeDtypeStruct((n, 1), jnp.float32),
        ],
    )


def _norm_body(acc_ref, den_ref, b_ref, o_ref, *, split16):
    d = den_ref[0, :] + den_ref[1, :]
    s = 1.0 / (d + 1e-16)
    if split16:
        a = jnp.concatenate([acc_ref[c] for c in range(8)], axis=1)
    else:
        a = acc_ref[0] + acc_ref[1]
    o_ref[...] = jnp.maximum(a * s[:, None] + b_ref[0, :][None, :], 0.0)


@functools.lru_cache(None)
def _norm_call(n, n_pad, split16):
    if split16:
        acc_spec = pl.BlockSpec((8, ROW_BLK, 16), lambda i: (0, i, 0))
        acc_shape = (8, n_pad, 16)
    else:
        acc_spec = pl.BlockSpec((2, ROW_BLK, DIM), lambda i: (0, i, 0))
        acc_shape = (2, n_pad, DIM)
    del acc_shape
    return pl.pallas_call(
        functools.partial(_norm_body, split16=split16),
        grid=(pl.cdiv(n, ROW_BLK),),
        in_specs=[
            acc_spec,
            pl.BlockSpec((2, ROW_BLK), lambda i: (0, i)),
            pl.BlockSpec((1, DIM), lambda i: (0, 0)),
        ],
        out_specs=pl.BlockSpec((ROW_BLK, DIM), lambda i: (i, 0)),
        out_shape=jax.ShapeDtypeStruct((n, DIM), jnp.float32),
    )


@functools.lru_cache(None)
def _edge_kernel(n_src, n_dst, n_dst_pad):
    mesh = plsc.VectorSubcoreMesh(core_axis_name="c", subcore_axis_name="s")
    dchunk = n_dst_pad // 16

    def body(als_hbm, ald_hbm, src2_hbm, dst2_hbm, w2_hbm, den_hbm,
             als_v, ald_v, src2_v, dst2_v, w2_v, den_sh, zero_v):
        cid = lax.axis_index("c")
        sid = lax.axis_index("s")
        wid = cid * 16 + sid
        pltpu.sync_copy(als_hbm, als_v)
        pltpu.sync_copy(ald_hbm, ald_v)
        pltpu.sync_copy(src2_hbm.at[pl.ds(wid * NBATCH, NBATCH)], src2_v)
        pltpu.sync_copy(dst2_hbm.at[pl.ds(wid * NBATCH, NBATCH)], dst2_v)

        zz = jnp.zeros((16,), jnp.float32)

        def zfill(i, carry):
            zero_v[pl.ds(i * 16, 16)] = zz
            return carry

        lax.fori_loop(0, dchunk // 16, zfill, 0)
        pltpu.sync_copy(zero_v, den_sh.at[pl.ds(sid * dchunk, dchunk)])
        plsc.subcore_barrier()

        base = wid * EPT
        lane = lax.iota(jnp.int32, 16)

        def ebody(b, carry):
            for k in range(8):
                sv = src2_v[b, pl.ds(k * 16, 16)]
                dv = dst2_v[b, pl.ds(k * 16, 16)]
                e = plsc.load_gather(als_v, [sv]) + plsc.load_gather(ald_v, [dv])
                e = jnp.where(e >= 0.0, e, 0.2 * e)
                w = jnp.exp(e)
                eid = base + b * 128 + k * 16 + lane
                w = jnp.where(eid < E, w, 0.0)
                w2_v[b, pl.ds(k * 16, 16)] = w
            pltpu.sync_copy(w2_v.at[b], den_sh.at[dst2_v.at[b]], add=True)
            return carry

        lax.fori_loop(0, NBATCH, ebody, 0)
        pltpu.sync_copy(w2_v, w2_hbm.at[pl.ds(wid * NBATCH, NBATCH)])
        plsc.subcore_barrier()
        pltpu.sync_copy(den_sh.at[pl.ds(sid * dchunk, dchunk)],
                        den_hbm.at[cid].at[pl.ds(sid * dchunk, dchunk)])

    return pl.kernel(
        body,
        out_type=[
            jax.ShapeDtypeStruct((EPAD // 128, 128), jnp.float32),
            jax.ShapeDtypeStruct((2, n_dst_pad), jnp.float32),
        ],
        mesh=mesh,
        compiler_params=pltpu.CompilerParams(needs_layout_passes=False),
        scratch_types=[
            pltpu.VMEM((n_src,), jnp.float32),
            pltpu.VMEM((n_dst,), jnp.float32),
            pltpu.VMEM((NBATCH, 128), jnp.int32),
            pltpu.VMEM((NBATCH, 128), jnp.int32),
            pltpu.VMEM((NBATCH, 128), jnp.float32),
            pltpu.VMEM_SHARED((n_dst_pad,), jnp.float32),
            pltpu.VMEM((dchunk,), jnp.float32),
        ],
    )


def _zero_acc(zrow, acc_sh, sid, rows_pt, ncol):
    def zloop(i, carry):
        pltpu.sync_copy(zrow, acc_sh.at[pl.ds(sid * rows_pt + i * 128, 128)])
        return carry
    lax.fori_loop(0, rows_pt // 128, zloop, 0)


def _fill_zrow(zrow, ncol):
    zz = jnp.zeros((16,), jnp.float32)

    def zfill(i, carry):
        for q in range(ncol // 16):
            zrow[i, pl.ds(q * 16, 16)] = zz
        return carry

    lax.fori_loop(0, 128, zfill, 0)


GRP = 16


@functools.lru_cache(None)
def _scatter128_kernel(n_src, n_dst_pad):
    mesh = plsc.VectorSubcoreMesh(core_axis_name="c", subcore_axis_name="s")
    rows_pt = n_dst_pad // 16

    def body(hs_hbm, src2_hbm, dst2_hbm, w2_hbm, acc_hbm,
             src2_v, dst2_v, w2_v, g0, g1, acc_sh, gsem0, gsem1):
        cid = lax.axis_index("c")
        sid = lax.axis_index("s")
        wid = cid * 16 + sid
        _fill_zrow(g0, DIM)
        _zero_acc(g0, acc_sh, sid, rows_pt, DIM)
        plsc.subcore_barrier()

        gbufs = (g0, g1)
        gsems = (gsem0, gsem1)

        def group(gi, carry):
            gbase = wid * NBATCH + gi * GRP
            pltpu.sync_copy(src2_hbm.at[pl.ds(gbase, GRP)], src2_v)
            pltpu.sync_copy(dst2_hbm.at[pl.ds(gbase, GRP)], dst2_v)
            pltpu.sync_copy(w2_hbm.at[pl.ds(gbase, GRP)], w2_v)
            pltpu.async_copy(hs_hbm.at[src2_v.at[0]], g0, gsem0)
            pltpu.async_copy(hs_hbm.at[src2_v.at[1]], g1, gsem1)

            def step(j, b):
                g = gbufs[j]
                sem = gsems[j]
                pltpu.make_async_copy(hs_hbm.at[src2_v.at[b]], g, sem).wait()
                for k in range(8):
                    wv = w2_v[b, pl.ds(k * 16, 16)]
                    for t in range(16):
                        ew = jnp.full((16,), wv[t])
                        row = k * 16 + t
                        for q in range(8):
                            g[row, pl.ds(q * 16, 16)] = g[row, pl.ds(q * 16, 16)] * ew
                pltpu.sync_copy(g, acc_sh.at[dst2_v.at[b]], add=True)

                @pl.when(b + 2 < GRP)
                def _():
                    pltpu.async_copy(hs_hbm.at[src2_v.at[b + 2]], g, sem)

            def sbody(i, carry2):
                step(0, 2 * i)
                step(1, 2 * i + 1)
                return carry2

            lax.fori_loop(0, GRP // 2, sbody, 0)
            return carry

        lax.fori_loop(0, NBATCH // GRP, group, 0)
        plsc.subcore_barrier()
        pltpu.sync_copy(acc_sh.at[pl.ds(sid * rows_pt, rows_pt)],
                        acc_hbm.at[cid].at[pl.ds(sid * rows_pt, rows_pt)])

    return pl.kernel(
        body,
        out_type=jax.ShapeDtypeStruct((2, n_dst_pad, DIM), jnp.float32),
        mesh=mesh,
        compiler_params=pltpu.CompilerParams(needs_layout_passes=False,
                                             use_tc_tiling_on_sc=False),
        scratch_types=[
            pltpu.VMEM((GRP, 128), jnp.int32),
            pltpu.VMEM((GRP, 128), jnp.int32),
            pltpu.VMEM((GRP, 128), jnp.float32),
            pltpu.VMEM((128, DIM), jnp.float32),
            pltpu.VMEM((128, DIM), jnp.float32),
            pltpu.VMEM_SHARED((n_dst_pad, DIM), jnp.float32),
            pltpu.SemaphoreType.DMA,
            pltpu.SemaphoreType.DMA,
        ],
    )


@functools.lru_cache(None)
def _scatter16_kernel(n_src, n_dst_pad):
    mesh = plsc.VectorSubcoreMesh(core_axis_name="c", subcore_axis_name="s")
    rows_pt = n_dst_pad // 16

    def body(hs8_hbm, src2_hbm, dst2_hbm, w2_hbm, acc_hbm,
             src2_v, dst2_v, w2_v, g0, g1, acc_sh, zrow, gsem0, gsem1):
        cid = lax.axis_index("c")
        sid = lax.axis_index("s")
        pltpu.sync_copy(src2_hbm.at[pl.ds(sid * NBATCH16, NBATCH16)], src2_v)
        pltpu.sync_copy(dst2_hbm.at[pl.ds(sid * NBATCH16, NBATCH16)], dst2_v)
        pltpu.sync_copy(w2_hbm.at[pl.ds(sid * NBATCH16, NBATCH16)], w2_v)
        _fill_zrow(zrow, 16)

        gbufs = (g0, g1)
        gsems = (gsem0, gsem1)

        for cc in range(4):
            chunk = cid * 4 + cc
            tab = hs8_hbm.at[chunk]
            _zero_acc(zrow, acc_sh, sid, rows_pt, 16)
            plsc.subcore_barrier()

            pltpu.async_copy(tab.at[src2_v.at[0]], g0, gsem0)
            pltpu.async_copy(tab.at[src2_v.at[1]], g1, gsem1)

            def step(j, b):
                g = gbufs[j]
                sem = gsems[j]
                pltpu.make_async_copy(tab.at[src2_v.at[b]], g, sem).wait()
                for k in range(8):
                    wv = w2_v[b, pl.ds(k * 16, 16)]
                    for t in range(16):
                        row = k * 16 + t
                        ew = jnp.full((16,), wv[t])
                        g[row, pl.ds(0, 16)] = g[row, pl.ds(0, 16)] * ew
                pltpu.sync_copy(g, acc_sh.at[dst2_v.at[b]], add=True)

                @pl.when(b + 2 < NBATCH16)
                def _():
                    pltpu.async_copy(tab.at[src2_v.at[b + 2]], g, sem)

            def sbody(i, carry):
                step(0, 2 * i)
                step(1, 2 * i + 1)
                return carry

            lax.fori_loop(0, NBATCH16 // 2, sbody, 0)
            plsc.subcore_barrier()
            pltpu.sync_copy(acc_sh.at[pl.ds(sid * rows_pt, rows_pt)],
                            acc_hbm.at[chunk].at[pl.ds(sid * rows_pt, rows_pt)])

    return pl.kernel(
        body,
        out_type=jax.ShapeDtypeStruct((8, n_dst_pad, 16), jnp.float32),
        mesh=mesh,
        compiler_params=pltpu.CompilerParams(needs_layout_passes=False,
                                             use_tc_tiling_on_sc=False),
        scratch_types=[
            pltpu.VMEM((NBATCH16, 128), jnp.int32),
            pltpu.VMEM((NBATCH16, 128), jnp.int32),
            pltpu.VMEM((NBATCH16, 128), jnp.float32),
            pltpu.VMEM((128, 16), jnp.float32),
            pltpu.VMEM((128, 16), jnp.float32),
            pltpu.VMEM_SHARED((n_dst_pad, 16), jnp.float32),
            pltpu.VMEM((128, 16), jnp.float32),
            pltpu.SemaphoreType.DMA,
            pltpu.SemaphoreType.DMA,
        ],
    )


def kernel(x_drug, x_protein, Wenc_d, benc_d, Wenc_p, benc_p, Wsrc, Wdst,
           Asrc, Adst, Bconv, edge_index_drug_binds_protein,
           edge_index_protein_rev_binds_drug):
    def prep_edges(ei, n_src, n_dst):
        src = ei[0].astype(jnp.int32)
        dst = ei[1].astype(jnp.int32)
        pad = jnp.arange(EPAD - E, dtype=jnp.int32)
        src = jnp.concatenate([src, pad % n_src]).reshape(EPAD // 128, 128)
        dst = jnp.concatenate([dst, pad % n_dst]).reshape(EPAD // 128, 128)
        return src, dst

    src_dp, dst_dp = prep_edges(edge_index_drug_binds_protein, ND, NP)
    src_pd, dst_pd = prep_edges(edge_index_protein_rev_binds_drug, NP, ND)

    hd = _enc_call(ND)(x_drug, Wenc_d, benc_d.reshape(1, DIM))
    hp = _enc_call(NP)(x_protein, Wenc_p, benc_p.reshape(1, DIM))

    for l in range(NLAYER):
        hs_dp8, als_dp2, ald_pd2 = _proj_call(ND, True)(
            hd, Wsrc[l, 0], Asrc[l, 0].reshape(1, DIM),
            Wdst[l, 1], Adst[l, 1].reshape(1, DIM))
        hs_pd, als_pd2, ald_dp2 = _proj_call(NP, False)(
            hp, Wsrc[l, 1], Asrc[l, 1].reshape(1, DIM),
            Wdst[l, 0], Adst[l, 0].reshape(1, DIM))

        w_dp, den_dp = _edge_kernel(ND, NP, NP_PAD)(
            als_dp2.reshape(ND), ald_dp2.reshape(NP), src_dp, dst_dp)
        w_pd, den_pd = _edge_kernel(NP, ND, ND_PAD)(
            als_pd2.reshape(NP), ald_pd2.reshape(ND), src_pd, dst_pd)

        acc_dp = _scatter16_kernel(ND, NP_PAD)(hs_dp8, src_dp, dst_dp, w_dp)
        acc_pd = _scatter128_kernel(NP, ND_PAD)(hs_pd, src_pd, dst_pd, w_pd)

        hp = _norm_call(NP, NP_PAD, True)(acc_dp, den_dp, Bconv[l, 0].reshape(1, DIM))
        hd = _norm_call(ND, ND_PAD, False)(acc_pd, den_pd, Bconv[l, 1].reshape(1, DIM))

    return hd, hp
